# baseline (speedup 1.0000x reference)
import functools

import jax
import jax.numpy as jnp
from jax.experimental import pallas as pl
from jax.experimental.pallas import tpu as pltpu


def _phrase_kernel(s1_ref, s2_ref, wenc_ref, benc_ref, w1_ref,
                   b1_ref, w2_ref, b2_ref, out_ref, *, odim, inv_l):
    acc1 = jnp.sum(s1_ref[...], axis=0)
    acc2 = jnp.sum(s2_ref[...], axis=0)

    wenc = wenc_ref[...] * inv_l
    benc = benc_ref[...]
    h1 = jnp.tanh(jnp.dot(acc1, wenc,
                          preferred_element_type=jnp.float32) + benc)
    h2 = jnp.tanh(jnp.dot(acc2, wenc,
                          preferred_element_type=jnp.float32) + benc)

    w1 = w1_ref[...]
    z = (jnp.dot(h1, w1[0 * odim:1 * odim, :],
                 preferred_element_type=jnp.float32)
         + jnp.dot(h2, w1[1 * odim:2 * odim, :],
                   preferred_element_type=jnp.float32)
         + jnp.dot(jnp.abs(h1 - h2), w1[2 * odim:3 * odim, :],
                   preferred_element_type=jnp.float32)
         + jnp.dot(h1 * h2, w1[3 * odim:4 * odim, :],
                   preferred_element_type=jnp.float32)
         + b1_ref[...])
    z = jnp.maximum(z, 0.0)

    logits = jnp.sum(z * w2_ref[...], axis=-1) + b2_ref[0, 0]
    out_ref[...] = (1.0 / (1.0 + jnp.exp(-logits)))[None, :]


def kernel(seq1, seq2, wenc, benc, w1, b1, w2, b2):
    L, B, idim = seq1.shape
    odim = wenc.shape[1]

    bt = B if B <= 128 else 128
    assert B % bt == 0
    nb = B // bt

    const = lambda shape: pl.BlockSpec(shape, lambda b: (0, 0))

    out = pl.pallas_call(
        functools.partial(_phrase_kernel, odim=odim, inv_l=1.0 / L),
        out_shape=jax.ShapeDtypeStruct((1, B), jnp.float32),
        grid=(nb,),
        in_specs=[
            pl.BlockSpec((L, bt, idim), lambda b: (0, b, 0)),
            pl.BlockSpec((L, bt, idim), lambda b: (0, b, 0)),
            const((idim, odim)),
            const((1, odim)),
            const((4 * odim, odim)),
            const((1, odim)),
            const((1, odim)),
            const((1, 1)),
        ],
        out_specs=pl.BlockSpec((1, bt), lambda b: (0, b)),
        compiler_params=pltpu.CompilerParams(
            dimension_semantics=("parallel",),
            vmem_limit_bytes=56 << 20),
    )(seq1, seq2, wenc, benc, w1, b1, w2.reshape(1, odim), b2)

    return out.reshape(B, 1)


# speedup vs baseline: 1.0779x; 1.0779x over previous
import functools

import jax
import jax.numpy as jnp
from jax.experimental import pallas as pl
from jax.experimental.pallas import tpu as pltpu


def _phrase_kernel(s1_ref, s2_ref, wenc_ref, benc_ref, w1_ref,
                   b1_ref, w2_ref, b2_ref, out_ref, *, odim, inv_l):
    acc1 = jnp.sum(s1_ref[...], axis=0)
    acc2 = jnp.sum(s2_ref[...], axis=0)

    wenc = wenc_ref[...] * inv_l
    benc = benc_ref[...]
    h1 = jnp.tanh(jnp.dot(acc1, wenc,
                          preferred_element_type=jnp.float32) + benc)
    h2 = jnp.tanh(jnp.dot(acc2, wenc,
                          preferred_element_type=jnp.float32) + benc)

    w1 = w1_ref[...]
    z = (jnp.dot(h1, w1[0 * odim:1 * odim, :],
                 preferred_element_type=jnp.float32)
         + jnp.dot(h2, w1[1 * odim:2 * odim, :],
                   preferred_element_type=jnp.float32)
         + jnp.dot(jnp.abs(h1 - h2), w1[2 * odim:3 * odim, :],
                   preferred_element_type=jnp.float32)
         + jnp.dot(h1 * h2, w1[3 * odim:4 * odim, :],
                   preferred_element_type=jnp.float32)
         + b1_ref[...])
    z = jnp.maximum(z, 0.0)

    logits = jnp.sum(z * w2_ref[...], axis=-1) + b2_ref[0, 0]
    out_ref[...] = (1.0 / (1.0 + jnp.exp(-logits)))[None, :]


def kernel(seq1, seq2, wenc, benc, w1, b1, w2, b2):
    L, B, idim = seq1.shape
    odim = wenc.shape[1]

    bt = B if B <= 256 else 256
    assert B % bt == 0
    nb = B // bt

    const = lambda shape: pl.BlockSpec(shape, lambda b: (0, 0))

    out = pl.pallas_call(
        functools.partial(_phrase_kernel, odim=odim, inv_l=1.0 / L),
        out_shape=jax.ShapeDtypeStruct((1, B), jnp.float32),
        grid=(nb,),
        in_specs=[
            pl.BlockSpec((L, bt, idim), lambda b: (0, b, 0)),
            pl.BlockSpec((L, bt, idim), lambda b: (0, b, 0)),
            const((idim, odim)),
            const((1, odim)),
            const((4 * odim, odim)),
            const((1, odim)),
            const((1, odim)),
            const((1, 1)),
        ],
        out_specs=pl.BlockSpec((1, bt), lambda b: (0, b)),
        compiler_params=pltpu.CompilerParams(
            dimension_semantics=("parallel",),
            vmem_limit_bytes=56 << 20),
    )(seq1, seq2, wenc, benc, w1, b1, w2.reshape(1, odim), b2)

    return out.reshape(B, 1)


# speedup vs baseline: 1.0785x; 1.0006x over previous
import functools

import jax
import jax.numpy as jnp
from jax.experimental import pallas as pl
from jax.experimental.pallas import tpu as pltpu


def _phrase_kernel(s1_ref, s2_ref, wenc_ref, benc_ref, w1_ref,
                   b1_ref, w2_ref, b2_ref, out_ref, *, odim, inv_l):
    acc1 = jnp.sum(s1_ref[...], axis=0)
    wenc = wenc_ref[...] * inv_l
    benc = benc_ref[...]
    h1 = jnp.tanh(jnp.dot(acc1, wenc,
                          preferred_element_type=jnp.float32) + benc)
    w1 = w1_ref[...]
    z1 = jnp.dot(h1, w1[0 * odim:1 * odim, :],
                 preferred_element_type=jnp.float32)

    acc2 = jnp.sum(s2_ref[...], axis=0)
    h2 = jnp.tanh(jnp.dot(acc2, wenc,
                          preferred_element_type=jnp.float32) + benc)

    z = (z1
         + jnp.dot(h2, w1[1 * odim:2 * odim, :],
                   preferred_element_type=jnp.float32)
         + jnp.dot(jnp.abs(h1 - h2), w1[2 * odim:3 * odim, :],
                   preferred_element_type=jnp.float32)
         + jnp.dot(h1 * h2, w1[3 * odim:4 * odim, :],
                   preferred_element_type=jnp.float32)
         + b1_ref[...])
    z = jnp.maximum(z, 0.0)

    logits = jnp.sum(z * w2_ref[...], axis=-1) + b2_ref[0, 0]
    out_ref[...] = (1.0 / (1.0 + jnp.exp(-logits)))[None, :]


def kernel(seq1, seq2, wenc, benc, w1, b1, w2, b2):
    L, B, idim = seq1.shape
    odim = wenc.shape[1]

    bt = B if B <= 256 else 256
    assert B % bt == 0
    nb = B // bt

    const = lambda shape: pl.BlockSpec(shape, lambda b: (0, 0))

    out = pl.pallas_call(
        functools.partial(_phrase_kernel, odim=odim, inv_l=1.0 / L),
        out_shape=jax.ShapeDtypeStruct((1, B), jnp.float32),
        grid=(nb,),
        in_specs=[
            pl.BlockSpec((L, bt, idim), lambda b: (0, b, 0)),
            pl.BlockSpec((L, bt, idim), lambda b: (0, b, 0)),
            const((idim, odim)),
            const((1, odim)),
            const((4 * odim, odim)),
            const((1, odim)),
            const((1, odim)),
            const((1, 1)),
        ],
        out_specs=pl.BlockSpec((1, bt), lambda b: (0, b)),
        compiler_params=pltpu.CompilerParams(
            dimension_semantics=("parallel",),
            vmem_limit_bytes=56 << 20),
    )(seq1, seq2, wenc, benc, w1, b1, w2.reshape(1, odim), b2)

    return out.reshape(B, 1)


# speedup vs baseline: 1.0800x; 1.0014x over previous
import functools

import jax
import jax.numpy as jnp
from jax.experimental import pallas as pl
from jax.experimental.pallas import tpu as pltpu


def _phrase_kernel(s1_ref, s2_ref, wenc_ref, benc_ref, w1_ref,
                   b1_ref, w2_ref, b2_ref, out_ref, *, odim, inv_l):
    acc1 = jnp.sum(s1_ref[...], axis=0)
    acc2 = jnp.sum(s2_ref[...], axis=0)

    wenc = wenc_ref[...] * inv_l
    benc = benc_ref[...]
    h1 = jnp.tanh(jnp.dot(acc1, wenc,
                          preferred_element_type=jnp.float32) + benc)
    h2 = jnp.tanh(jnp.dot(acc2, wenc,
                          preferred_element_type=jnp.float32) + benc)

    w1 = w1_ref[...]
    z = (jnp.dot(h1, w1[0 * odim:1 * odim, :],
                 preferred_element_type=jnp.float32)
         + jnp.dot(h2, w1[1 * odim:2 * odim, :],
                   preferred_element_type=jnp.float32)
         + jnp.dot(jnp.abs(h1 - h2), w1[2 * odim:3 * odim, :],
                   preferred_element_type=jnp.float32)
         + jnp.dot(h1 * h2, w1[3 * odim:4 * odim, :],
                   preferred_element_type=jnp.float32)
         + b1_ref[...])
    z = jnp.maximum(z, 0.0)

    logits = jnp.sum(z * w2_ref[...], axis=-1) + b2_ref[0, 0]
    out_ref[...] = (1.0 / (1.0 + jnp.exp(-logits)))[None, :]


def kernel(seq1, seq2, wenc, benc, w1, b1, w2, b2):
    L, B, idim = seq1.shape
    odim = wenc.shape[1]

    bt = B if B <= 256 else 256
    assert B % bt == 0
    nb = B // bt

    const = lambda shape: pl.BlockSpec(shape, lambda b: (0, 0))

    out = pl.pallas_call(
        functools.partial(_phrase_kernel, odim=odim, inv_l=1.0 / L),
        out_shape=jax.ShapeDtypeStruct((1, B), jnp.float32),
        grid=(nb,),
        in_specs=[
            pl.BlockSpec((L, bt, idim), lambda b: (0, b, 0)),
            pl.BlockSpec((L, bt, idim), lambda b: (0, b, 0)),
            const((idim, odim)),
            const((1, odim)),
            const((4 * odim, odim)),
            const((1, odim)),
            const((1, odim)),
            const((1, 1)),
        ],
        out_specs=pl.BlockSpec((1, bt), lambda b: (0, b)),
        compiler_params=pltpu.CompilerParams(
            dimension_semantics=("parallel",),
            vmem_limit_bytes=56 << 20),
    )(seq1, seq2, wenc, benc, w1, b1, w2.reshape(1, odim), b2)

    return out.reshape(B, 1)


# speedup vs baseline: 1.0823x; 1.0021x over previous
import functools

import jax
import jax.numpy as jnp
from jax.experimental import pallas as pl
from jax.experimental.pallas import tpu as pltpu


def _phrase_kernel(s1_ref, s2_ref, wenc_ref, benc_ref, w1_ref,
                   b1_ref, w2_ref, b2_ref, out_ref, *, odim, inv_l):
    acc1 = jnp.sum(s1_ref[...], axis=0)
    acc2 = jnp.sum(s2_ref[...], axis=0)

    wenc = wenc_ref[...] * inv_l
    benc = benc_ref[...]
    h1 = jnp.tanh(jnp.dot(acc1, wenc,
                          preferred_element_type=jnp.float32) + benc)
    h2 = jnp.tanh(jnp.dot(acc2, wenc,
                          preferred_element_type=jnp.float32) + benc)

    w1 = w1_ref[...]
    z = (jnp.dot(h1, w1[0 * odim:1 * odim, :],
                 preferred_element_type=jnp.float32)
         + jnp.dot(h2, w1[1 * odim:2 * odim, :],
                   preferred_element_type=jnp.float32)
         + jnp.dot(jnp.abs(h1 - h2), w1[2 * odim:3 * odim, :],
                   preferred_element_type=jnp.float32)
         + jnp.dot(h1 * h2, w1[3 * odim:4 * odim, :],
                   preferred_element_type=jnp.float32)
         + b1_ref[...])
    z = jnp.maximum(z, 0.0)

    logits = jnp.sum(z * w2_ref[...], axis=-1) + b2_ref[0, 0]
    out_ref[...] = (1.0 / (1.0 + jnp.exp(-logits)))[None, :]


def kernel(seq1, seq2, wenc, benc, w1, b1, w2, b2):
    L, B, idim = seq1.shape
    odim = wenc.shape[1]

    bt = B if B <= 256 else 256
    assert B % bt == 0
    nb = B // bt

    const = lambda shape: pl.BlockSpec(shape, lambda b: (0, 0))

    out = pl.pallas_call(
        functools.partial(_phrase_kernel, odim=odim, inv_l=1.0 / L),
        out_shape=jax.ShapeDtypeStruct((1, B), jnp.float32),
        grid=(nb,),
        in_specs=[
            pl.BlockSpec((L, bt, idim), lambda b: (0, b, 0)),
            pl.BlockSpec((L, bt, idim), lambda b: (0, b, 0)),
            const((idim, odim)),
            const((1, odim)),
            const((4 * odim, odim)),
            const((1, odim)),
            const((1, odim)),
            const((1, 1)),
        ],
        out_specs=pl.BlockSpec((1, bt), lambda b: (0, b)),
        compiler_params=pltpu.CompilerParams(
            dimension_semantics=("parallel",)),
    )(seq1, seq2, wenc, benc, w1, b1, w2.reshape(1, odim), b2)

    return out.reshape(B, 1)
